# baseline (speedup 1.0000x reference)
import functools

import jax
import jax.numpy as jnp
from jax import lax
from jax.experimental import pallas as pl
from jax.experimental.pallas import tpu as pltpu
from jax.experimental.pallas import tpu_sc as plsc

NC = 2
NS = 16
LANES = 16
CHUNK = 128

_mesh = lambda: plsc.VectorSubcoreMesh(core_axis_name="c", subcore_axis_name="s")


def _round_up(a, b):
    return (a + b - 1) // b * b


def _make_deg_kernel(np_, ep):
    edges_per_tile = ep // (NC * NS)
    rows_per_tile = edges_per_tile // CHUNK
    n_per_tile = np_ // NS
    GRP = 8

    def body(dst_hbm, ew_hbm, out_hbm, dstp, ewp, zv, deg_sh, sd):
        c = lax.axis_index("c")
        s = lax.axis_index("s")
        trow = (c * NS + s) * rows_per_tile

        def zloop(i, _):
            zv[pl.ds(i * LANES, LANES)] = jnp.zeros((LANES,), jnp.float32)
            return _
        lax.fori_loop(0, n_per_tile // LANES, zloop, None)
        pltpu.sync_copy(zv, deg_sh.at[pl.ds(s * n_per_tile, n_per_tile)])

        pltpu.sync_copy(dst_hbm.at[pl.ds(trow, rows_per_tile)], dstp)
        pltpu.sync_copy(ew_hbm.at[pl.ds(trow, rows_per_tile)], ewp)
        plsc.subcore_barrier()

        def grp_loop(g, _):
            for u in range(GRP):
                pltpu.async_copy(ewp.at[g * GRP + u], deg_sh.at[dstp.at[g * GRP + u]],
                                 sd, add=True)
            for u in range(GRP):
                pltpu.make_async_copy(ewp.at[0], deg_sh.at[dstp.at[0]], sd).wait()
            return _
        lax.fori_loop(0, rows_per_tile // GRP, grp_loop, None)
        plsc.subcore_barrier()

        pltpu.sync_copy(deg_sh.at[pl.ds(s * n_per_tile, n_per_tile)],
                        out_hbm.at[c, pl.ds(s * n_per_tile, n_per_tile)])

    return pl.kernel(
        body,
        out_type=jax.ShapeDtypeStruct((NC, np_), jnp.float32),
        mesh=_mesh(),
        scratch_types=[
            pltpu.VMEM((rows_per_tile, CHUNK), jnp.int32),
            pltpu.VMEM((rows_per_tile, CHUNK), jnp.float32),
            pltpu.VMEM((n_per_tile,), jnp.float32),
            pltpu.VMEM_SHARED((np_,), jnp.float32),
            pltpu.SemaphoreType.DMA,
        ],
    )


PGC = 8
AGG_K0 = 160
AGG_K1 = 0


def _make_agg_kernel(np_, ep, d, k0=None, k1=None):
    total_chunks = ep // CHUNK
    if k0 is None:
        k0 = total_chunks // (2 * NS)
    if k1 is None:
        k1 = total_chunks // NS - k0
    assert NS * (k0 + k1) == total_chunks and k0 % (2 * PGC) == 0 and k1 % (2 * PGC) == 0, (k0, k1)
    n_per_tile = np_ // NS

    def body(y_hbm, src_hbm, dst_hbm, ew_hbm, out_hbm, *scratch):
        sp = scratch[0:2]
        dp = scratch[2:4]
        wp = scratch[4:6]
        rows = scratch[6:8]
        acc_sh = scratch[8]
        si = scratch[9:11]
        sg = scratch[11:13]

        c = lax.axis_index("c")
        s = lax.axis_index("s")
        trow = jnp.where(c == 0, s * k0, NS * k0 + s * k1)
        n_pairs = jnp.where(c == 0, k0 // (2 * PGC), k1 // (2 * PGC))

        def fire_page(prow, pb):
            pltpu.async_copy(src_hbm.at[pl.ds(prow, PGC)], sp[pb], si[pb])
            pltpu.async_copy(dst_hbm.at[pl.ds(prow, PGC)], dp[pb], si[pb])
            pltpu.async_copy(ew_hbm.at[pl.ds(prow, PGC)], wp[pb], si[pb])

        def wait_page(pb):
            pltpu.make_async_copy(src_hbm.at[pl.ds(0, PGC)], sp[pb], si[pb]).wait()
            pltpu.make_async_copy(dst_hbm.at[pl.ds(0, PGC)], dp[pb], si[pb]).wait()
            pltpu.make_async_copy(ew_hbm.at[pl.ds(0, PGC)], wp[pb], si[pb]).wait()

        def fire_gather(j, pb, rb):
            pltpu.async_copy(y_hbm.at[sp[pb].at[j]], rows[rb], sg[rb])

        def wait_gather(rb):
            pltpu.make_async_copy(y_hbm.at[sp[0].at[0]], rows[rb], sg[rb]).wait()

        def scale(j, pb, rb):
            def grp(g, _):
                wv = wp[pb][j, pl.ds(g * LANES, LANES)]
                for i in range(LANES):
                    w = wv[i]
                    row = g * LANES + i
                    for jj in range(d // LANES):
                        sl = rows[rb][row, pl.ds(jj * LANES, LANES)]
                        rows[rb][row, pl.ds(jj * LANES, LANES)] = sl * w
                return _
            lax.fori_loop(0, CHUNK // LANES, grp, None)

        def scatter(j, pb, rb):
            pltpu.sync_copy(rows[rb], acc_sh.at[dp[pb].at[j]], add=True)

        def zloop(i, _):
            for jj in range(d // LANES):
                rows[0][i, pl.ds(jj * LANES, LANES)] = jnp.zeros((LANES,), jnp.float32)
            return _
        lax.fori_loop(0, CHUNK, zloop, None)
        for b in range(n_per_tile // CHUNK):
            pltpu.sync_copy(rows[0], acc_sh.at[pl.ds(s * n_per_tile + b * CHUNK, CHUNK)])
        plsc.subcore_barrier()


        def do_page(prow, pb, pair_i, is_b):
            if not is_b:
                fire_page(prow + PGC, 1 - pb)
            else:
                @pl.when(pair_i <= n_pairs - 2)
                def _f():
                    fire_page(prow + PGC, 1 - pb)
            def cpair(jj, _):
                j0 = 2 * jj
                fire_gather(j0 + 1, pb, 1)
                wait_gather(0)
                scale(j0, pb, 0)
                scatter(j0, pb, 0)
                fire_gather(j0 + 2, pb, 0)
                wait_gather(1)
                scale(j0 + 1, pb, 1)
                scatter(j0 + 1, pb, 1)
                return _
            lax.fori_loop(0, PGC // 2 - 1, cpair, None)
            fire_gather(PGC - 1, pb, 1)
            wait_gather(0)
            scale(PGC - 2, pb, 0)
            scatter(PGC - 2, pb, 0)
            if not is_b:
                wait_page(1 - pb)
                fire_gather(0, 1 - pb, 0)
            else:
                @pl.when(pair_i <= n_pairs - 2)
                def _g():
                    wait_page(1 - pb)
                    fire_gather(0, 1 - pb, 0)
            wait_gather(1)
            scale(PGC - 1, pb, 1)
            scatter(PGC - 1, pb, 1)

        def pair_loop(pair_i, _):
            prow = trow + pair_i * 2 * PGC
            do_page(prow, 0, pair_i, False)
            do_page(prow + PGC, 1, pair_i, True)
            return _

        @pl.when(n_pairs > 0)
        def _edge_work():
            fire_page(trow, 0)
            wait_page(0)
            fire_gather(0, 0, 0)
            lax.fori_loop(0, n_pairs, pair_loop, None)
        plsc.subcore_barrier()

        pltpu.sync_copy(acc_sh.at[pl.ds(s * n_per_tile, n_per_tile)],
                        out_hbm.at[c, pl.ds(s * n_per_tile, n_per_tile)])

    return pl.kernel(
        body,
        out_type=jax.ShapeDtypeStruct((NC, np_, d), jnp.float32),
        mesh=_mesh(),
        scratch_types=(
            [pltpu.VMEM((PGC, CHUNK), jnp.int32) for _ in range(4)]
            + [pltpu.VMEM((PGC, CHUNK), jnp.float32) for _ in range(2)]
            + [pltpu.VMEM((CHUNK, d), jnp.float32) for _ in range(2)]
            + [pltpu.VMEM_SHARED((np_, d), jnp.float32)]
            + [pltpu.SemaphoreType.DMA for _ in range(4)]
        ),
    )


def _prep_body(degp_ref, x_ref, dinv_ref, y_ref):
    deg = degp_ref[0] + degp_ref[1] + 1.0
    dv = jnp.where(deg > 0, lax.rsqrt(deg), 0.0)
    dinv_ref[...] = dv
    y_ref[...] = dv * x_ref[...]


def _mid_body(aggp_ref, x_ref, dinv_ref, w1_ref, b1_ref, w2_ref, t_ref, y2_ref):
    dv = dinv_ref[...]
    ax = dv * (aggp_ref[0] + aggp_ref[1]) + (dv * dv) * x_ref[...]
    h = jnp.maximum(
        jnp.dot(ax, w1_ref[...], preferred_element_type=jnp.float32) + b1_ref[...],
        0.0)
    t = jnp.dot(h, w2_ref[...], preferred_element_type=jnp.float32)
    t_ref[...] = t
    y2_ref[...] = dv * t


def _final_body(aggp_ref, t_ref, dinv_ref, b2_ref, out_ref):
    dv = dinv_ref[...]
    out_ref[...] = (dv * (aggp_ref[0] + aggp_ref[1])
                    + (dv * dv) * t_ref[...] + b2_ref[...])


def kernel(x, edge_index, edge_weight, W1, b1, W2, b2):
    n, d_in = x.shape
    d_hid = W1.shape[1]
    d_out = W2.shape[1]
    e = edge_weight.shape[0]

    np_ = _round_up(n, NS * CHUNK)
    ep = _round_up(e, NC * NS * CHUNK * 2 * PGC)

    src = edge_index[0].astype(jnp.int32)
    dst = edge_index[1].astype(jnp.int32)
    ew = edge_weight.astype(jnp.float32)
    pad_e = ep - e
    src = jnp.concatenate([src, jnp.zeros((pad_e,), jnp.int32)]).reshape(ep // CHUNK, CHUNK)
    dst = jnp.concatenate([dst, jnp.zeros((pad_e,), jnp.int32)]).reshape(ep // CHUNK, CHUNK)
    ew = jnp.concatenate([ew, jnp.zeros((pad_e,), jnp.float32)]).reshape(ep // CHUNK, CHUNK)
    xp = jnp.concatenate([x, jnp.zeros((np_ - n, d_in), x.dtype)])

    degp = _make_deg_kernel(np_, ep)(dst, ew)

    r = 512
    grid = (np_ // r,)
    dinv, y = pl.pallas_call(
        _prep_body,
        grid=grid,
        in_specs=[
            pl.BlockSpec((NC, r, 1), lambda i: (0, i, 0)),
            pl.BlockSpec((r, d_in), lambda i: (i, 0)),
        ],
        out_specs=[
            pl.BlockSpec((r, 1), lambda i: (i, 0)),
            pl.BlockSpec((r, d_in), lambda i: (i, 0)),
        ],
        out_shape=[
            jax.ShapeDtypeStruct((np_, 1), jnp.float32),
            jax.ShapeDtypeStruct((np_, d_in), jnp.float32),
        ],
    )(degp.reshape(NC, np_, 1), xp)

    agg1 = _make_agg_kernel(np_, ep, d_in, AGG_K0, AGG_K1)(y, src, dst, ew)

    t, y2 = pl.pallas_call(
        _mid_body,
        grid=grid,
        in_specs=[
            pl.BlockSpec((NC, r, d_in), lambda i: (0, i, 0)),
            pl.BlockSpec((r, d_in), lambda i: (i, 0)),
            pl.BlockSpec((r, 1), lambda i: (i, 0)),
            pl.BlockSpec((d_in, d_hid), lambda i: (0, 0)),
            pl.BlockSpec((1, d_hid), lambda i: (0, 0)),
            pl.BlockSpec((d_hid, d_out), lambda i: (0, 0)),
        ],
        out_specs=[
            pl.BlockSpec((r, d_out), lambda i: (i, 0)),
            pl.BlockSpec((r, d_out), lambda i: (i, 0)),
        ],
        out_shape=[
            jax.ShapeDtypeStruct((np_, d_out), jnp.float32),
            jax.ShapeDtypeStruct((np_, d_out), jnp.float32),
        ],
    )(agg1, xp, dinv, W1, b1.reshape(1, d_hid), W2)

    agg2 = _make_agg_kernel(np_, ep, d_out, AGG_K0, AGG_K1)(y2, src, dst, ew)

    out = pl.pallas_call(
        _final_body,
        grid=grid,
        in_specs=[
            pl.BlockSpec((NC, r, d_out), lambda i: (0, i, 0)),
            pl.BlockSpec((r, d_out), lambda i: (i, 0)),
            pl.BlockSpec((r, 1), lambda i: (i, 0)),
            pl.BlockSpec((1, d_out), lambda i: (0, 0)),
        ],
        out_specs=pl.BlockSpec((r, d_out), lambda i: (i, 0)),
        out_shape=jax.ShapeDtypeStruct((np_, d_out), jnp.float32),
    )(agg2, t, dinv, b2.reshape(1, d_out))

    return out[:n]


# speedup vs baseline: 3.6008x; 3.6008x over previous
import functools

import jax
import jax.numpy as jnp
from jax import lax
from jax.experimental import pallas as pl
from jax.experimental.pallas import tpu as pltpu
from jax.experimental.pallas import tpu_sc as plsc

NC = 2
NS = 16
LANES = 16
CHUNK = 128

_mesh = lambda: plsc.VectorSubcoreMesh(core_axis_name="c", subcore_axis_name="s")


def _round_up(a, b):
    return (a + b - 1) // b * b


def _make_deg_kernel(np_, ep):
    edges_per_tile = ep // (NC * NS)
    rows_per_tile = edges_per_tile // CHUNK
    n_per_tile = np_ // NS
    GRP = 8

    def body(dst_hbm, ew_hbm, out_hbm, dstp, ewp, zv, deg_sh, sd):
        c = lax.axis_index("c")
        s = lax.axis_index("s")
        trow = (c * NS + s) * rows_per_tile

        def zloop(i, _):
            zv[pl.ds(i * LANES, LANES)] = jnp.zeros((LANES,), jnp.float32)
            return _
        lax.fori_loop(0, n_per_tile // LANES, zloop, None)
        pltpu.sync_copy(zv, deg_sh.at[pl.ds(s * n_per_tile, n_per_tile)])

        pltpu.sync_copy(dst_hbm.at[pl.ds(trow, rows_per_tile)], dstp)
        pltpu.sync_copy(ew_hbm.at[pl.ds(trow, rows_per_tile)], ewp)
        plsc.subcore_barrier()

        def grp_loop(g, _):
            for u in range(GRP):
                pltpu.async_copy(ewp.at[g * GRP + u], deg_sh.at[dstp.at[g * GRP + u]],
                                 sd, add=True)
            for u in range(GRP):
                pltpu.make_async_copy(ewp.at[0], deg_sh.at[dstp.at[0]], sd).wait()
            return _
        lax.fori_loop(0, rows_per_tile // GRP, grp_loop, None)
        plsc.subcore_barrier()

        pltpu.sync_copy(deg_sh.at[pl.ds(s * n_per_tile, n_per_tile)],
                        out_hbm.at[c, pl.ds(s * n_per_tile, n_per_tile)])

    return pl.kernel(
        body,
        out_type=jax.ShapeDtypeStruct((NC, np_), jnp.float32),
        mesh=_mesh(),
        scratch_types=[
            pltpu.VMEM((rows_per_tile, CHUNK), jnp.int32),
            pltpu.VMEM((rows_per_tile, CHUNK), jnp.float32),
            pltpu.VMEM((n_per_tile,), jnp.float32),
            pltpu.VMEM_SHARED((np_,), jnp.float32),
            pltpu.SemaphoreType.DMA,
        ],
    )


PGC = 8
AGG_K0 = 80
AGG_K1 = 80


def _make_agg_kernel(np_, ep, d, k0=None, k1=None):
    total_chunks = ep // CHUNK
    if k0 is None:
        k0 = total_chunks // (2 * NS)
    if k1 is None:
        k1 = total_chunks // NS - k0
    assert NS * (k0 + k1) == total_chunks and k0 % (2 * PGC) == 0 and k1 % (2 * PGC) == 0, (k0, k1)
    n_per_tile = np_ // NS

    def body(y_hbm, src_hbm, dst_hbm, ew_hbm, out_hbm, *scratch):
        sp = scratch[0:2]
        dp = scratch[2:4]
        wp = scratch[4:6]
        rows = scratch[6:8]
        acc_sh = scratch[8]
        si = scratch[9:11]
        sg = scratch[11:13]

        c = lax.axis_index("c")
        s = lax.axis_index("s")
        trow = jnp.where(c == 0, s * k0, NS * k0 + s * k1)
        n_pairs = jnp.where(c == 0, k0 // (2 * PGC), k1 // (2 * PGC))

        def fire_page(prow, pb):
            pltpu.async_copy(src_hbm.at[pl.ds(prow, PGC)], sp[pb], si[pb])
            pltpu.async_copy(dst_hbm.at[pl.ds(prow, PGC)], dp[pb], si[pb])
            pltpu.async_copy(ew_hbm.at[pl.ds(prow, PGC)], wp[pb], si[pb])

        def wait_page(pb):
            pltpu.make_async_copy(src_hbm.at[pl.ds(0, PGC)], sp[pb], si[pb]).wait()
            pltpu.make_async_copy(dst_hbm.at[pl.ds(0, PGC)], dp[pb], si[pb]).wait()
            pltpu.make_async_copy(ew_hbm.at[pl.ds(0, PGC)], wp[pb], si[pb]).wait()

        def fire_gather(j, pb, rb):
            pltpu.async_copy(y_hbm.at[sp[pb].at[j]], rows[rb], sg[rb])

        def wait_gather(rb):
            pltpu.make_async_copy(y_hbm.at[sp[0].at[0]], rows[rb], sg[rb]).wait()

        def scale(j, pb, rb):
            def grp(g, _):
                wv = wp[pb][j, pl.ds(g * LANES, LANES)]
                for i in range(LANES):
                    w = wv[i]
                    row = g * LANES + i
                    for jj in range(d // LANES):
                        sl = rows[rb][row, pl.ds(jj * LANES, LANES)]
                        rows[rb][row, pl.ds(jj * LANES, LANES)] = sl * w
                return _
            lax.fori_loop(0, CHUNK // LANES, grp, None)

        def scatter(j, pb, rb):
            pltpu.sync_copy(rows[rb], acc_sh.at[dp[pb].at[j]], add=True)

        def zloop(i, _):
            for jj in range(d // LANES):
                rows[0][i, pl.ds(jj * LANES, LANES)] = jnp.zeros((LANES,), jnp.float32)
            return _
        lax.fori_loop(0, CHUNK, zloop, None)
        for b in range(n_per_tile // CHUNK):
            pltpu.sync_copy(rows[0], acc_sh.at[pl.ds(s * n_per_tile + b * CHUNK, CHUNK)])
        plsc.subcore_barrier()


        def do_page(prow, pb, pair_i, is_b):
            if not is_b:
                fire_page(prow + PGC, 1 - pb)
            else:
                @pl.when(pair_i <= n_pairs - 2)
                def _f():
                    fire_page(prow + PGC, 1 - pb)
            def cpair(jj, _):
                j0 = 2 * jj
                fire_gather(j0 + 1, pb, 1)
                wait_gather(0)
                scale(j0, pb, 0)
                scatter(j0, pb, 0)
                fire_gather(j0 + 2, pb, 0)
                wait_gather(1)
                scale(j0 + 1, pb, 1)
                scatter(j0 + 1, pb, 1)
                return _
            lax.fori_loop(0, PGC // 2 - 1, cpair, None)
            fire_gather(PGC - 1, pb, 1)
            wait_gather(0)
            scale(PGC - 2, pb, 0)
            scatter(PGC - 2, pb, 0)
            if not is_b:
                wait_page(1 - pb)
                fire_gather(0, 1 - pb, 0)
            else:
                @pl.when(pair_i <= n_pairs - 2)
                def _g():
                    wait_page(1 - pb)
                    fire_gather(0, 1 - pb, 0)
            wait_gather(1)
            scale(PGC - 1, pb, 1)
            scatter(PGC - 1, pb, 1)

        def pair_loop(pair_i, _):
            prow = trow + pair_i * 2 * PGC
            do_page(prow, 0, pair_i, False)
            do_page(prow + PGC, 1, pair_i, True)
            return _

        @pl.when(n_pairs > 0)
        def _edge_work():
            fire_page(trow, 0)
            wait_page(0)
            fire_gather(0, 0, 0)
            lax.fori_loop(0, n_pairs, pair_loop, None)
        plsc.subcore_barrier()

        pltpu.sync_copy(acc_sh.at[pl.ds(s * n_per_tile, n_per_tile)],
                        out_hbm.at[c, pl.ds(s * n_per_tile, n_per_tile)])

    return pl.kernel(
        body,
        out_type=jax.ShapeDtypeStruct((NC, np_, d), jnp.float32),
        mesh=_mesh(),
        scratch_types=(
            [pltpu.VMEM((PGC, CHUNK), jnp.int32) for _ in range(4)]
            + [pltpu.VMEM((PGC, CHUNK), jnp.float32) for _ in range(2)]
            + [pltpu.VMEM((CHUNK, d), jnp.float32) for _ in range(2)]
            + [pltpu.VMEM_SHARED((np_, d), jnp.float32)]
            + [pltpu.SemaphoreType.DMA for _ in range(4)]
        ),
    )


def _prep_body(degp_ref, x_ref, dinv_ref, y_ref):
    deg = degp_ref[0] + degp_ref[1] + 1.0
    dv = jnp.where(deg > 0, lax.rsqrt(deg), 0.0)
    dinv_ref[...] = dv
    y_ref[...] = dv * x_ref[...]


def _mid_body(aggp_ref, x_ref, dinv_ref, w1_ref, b1_ref, w2_ref, t_ref, y2_ref):
    dv = dinv_ref[...]
    ax = dv * (aggp_ref[0] + aggp_ref[1]) + (dv * dv) * x_ref[...]
    h = jnp.maximum(
        jnp.dot(ax, w1_ref[...], preferred_element_type=jnp.float32) + b1_ref[...],
        0.0)
    t = jnp.dot(h, w2_ref[...], preferred_element_type=jnp.float32)
    t_ref[...] = t
    y2_ref[...] = dv * t


def _final_body(aggp_ref, t_ref, dinv_ref, b2_ref, out_ref):
    dv = dinv_ref[...]
    out_ref[...] = (dv * (aggp_ref[0] + aggp_ref[1])
                    + (dv * dv) * t_ref[...] + b2_ref[...])


def kernel(x, edge_index, edge_weight, W1, b1, W2, b2):
    n, d_in = x.shape
    d_hid = W1.shape[1]
    d_out = W2.shape[1]
    e = edge_weight.shape[0]

    np_ = _round_up(n, NS * CHUNK)
    ep = _round_up(e, NC * NS * CHUNK * 2 * PGC)

    src = edge_index[0].astype(jnp.int32)
    dst = edge_index[1].astype(jnp.int32)
    ew = edge_weight.astype(jnp.float32)
    pad_e = ep - e
    pad_idx = jnp.arange(pad_e, dtype=jnp.int32)
    src = jnp.concatenate([src, pad_idx % n]).reshape(ep // CHUNK, CHUNK)
    dst = jnp.concatenate([dst, pad_idx % np_]).reshape(ep // CHUNK, CHUNK)
    ew = jnp.concatenate([ew, jnp.zeros((pad_e,), jnp.float32)]).reshape(ep // CHUNK, CHUNK)
    xp = jnp.concatenate([x, jnp.zeros((np_ - n, d_in), x.dtype)])

    degp = _make_deg_kernel(np_, ep)(dst, ew)

    r = 512
    grid = (np_ // r,)
    dinv, y = pl.pallas_call(
        _prep_body,
        grid=grid,
        in_specs=[
            pl.BlockSpec((NC, r, 1), lambda i: (0, i, 0)),
            pl.BlockSpec((r, d_in), lambda i: (i, 0)),
        ],
        out_specs=[
            pl.BlockSpec((r, 1), lambda i: (i, 0)),
            pl.BlockSpec((r, d_in), lambda i: (i, 0)),
        ],
        out_shape=[
            jax.ShapeDtypeStruct((np_, 1), jnp.float32),
            jax.ShapeDtypeStruct((np_, d_in), jnp.float32),
        ],
    )(degp.reshape(NC, np_, 1), xp)

    agg1 = _make_agg_kernel(np_, ep, d_in, AGG_K0, AGG_K1)(y, src, dst, ew)

    t, y2 = pl.pallas_call(
        _mid_body,
        grid=grid,
        in_specs=[
            pl.BlockSpec((NC, r, d_in), lambda i: (0, i, 0)),
            pl.BlockSpec((r, d_in), lambda i: (i, 0)),
            pl.BlockSpec((r, 1), lambda i: (i, 0)),
            pl.BlockSpec((d_in, d_hid), lambda i: (0, 0)),
            pl.BlockSpec((1, d_hid), lambda i: (0, 0)),
            pl.BlockSpec((d_hid, d_out), lambda i: (0, 0)),
        ],
        out_specs=[
            pl.BlockSpec((r, d_out), lambda i: (i, 0)),
            pl.BlockSpec((r, d_out), lambda i: (i, 0)),
        ],
        out_shape=[
            jax.ShapeDtypeStruct((np_, d_out), jnp.float32),
            jax.ShapeDtypeStruct((np_, d_out), jnp.float32),
        ],
    )(agg1, xp, dinv, W1, b1.reshape(1, d_hid), W2)

    agg2 = _make_agg_kernel(np_, ep, d_out, AGG_K0, AGG_K1)(y2, src, dst, ew)

    out = pl.pallas_call(
        _final_body,
        grid=grid,
        in_specs=[
            pl.BlockSpec((NC, r, d_out), lambda i: (0, i, 0)),
            pl.BlockSpec((r, d_out), lambda i: (i, 0)),
            pl.BlockSpec((r, 1), lambda i: (i, 0)),
            pl.BlockSpec((1, d_out), lambda i: (0, 0)),
        ],
        out_specs=pl.BlockSpec((r, d_out), lambda i: (i, 0)),
        out_shape=jax.ShapeDtypeStruct((np_, d_out), jnp.float32),
    )(agg2, t, dinv, b2.reshape(1, d_out))

    return out[:n]


# speedup vs baseline: 3.6030x; 1.0006x over previous
import jax
import jax.numpy as jnp
from jax import lax
from jax.experimental import pallas as pl
from jax.experimental.pallas import tpu as pltpu
from jax.experimental.pallas import tpu_sc as plsc

NC = 2
NS = 16
LANES = 16
CHUNK = 128

_mesh = lambda: plsc.VectorSubcoreMesh(core_axis_name="c", subcore_axis_name="s")


def _round_up(a, b):
    return (a + b - 1) // b * b


def _make_deg_kernel(np_, ep):
    edges_per_tile = ep // (NC * NS)
    rows_per_tile = edges_per_tile // CHUNK
    n_per_tile = np_ // NS
    GRP = 8

    def body(dst_hbm, ew_hbm, out_hbm, dstp, ewp, zv, deg_sh, sd):
        c = lax.axis_index("c")
        s = lax.axis_index("s")
        trow = (c * NS + s) * rows_per_tile

        def zloop(i, _):
            zv[pl.ds(i * LANES, LANES)] = jnp.zeros((LANES,), jnp.float32)
            return _
        lax.fori_loop(0, n_per_tile // LANES, zloop, None)
        pltpu.sync_copy(zv, deg_sh.at[pl.ds(s * n_per_tile, n_per_tile)])

        pltpu.sync_copy(dst_hbm.at[pl.ds(trow, rows_per_tile)], dstp)
        pltpu.sync_copy(ew_hbm.at[pl.ds(trow, rows_per_tile)], ewp)
        plsc.subcore_barrier()

        def grp_loop(g, _):
            for u in range(GRP):
                pltpu.async_copy(ewp.at[g * GRP + u], deg_sh.at[dstp.at[g * GRP + u]],
                                 sd, add=True)
            for u in range(GRP):
                pltpu.make_async_copy(ewp.at[0], deg_sh.at[dstp.at[0]], sd).wait()
            return _
        lax.fori_loop(0, rows_per_tile // GRP, grp_loop, None)
        plsc.subcore_barrier()

        pltpu.sync_copy(deg_sh.at[pl.ds(s * n_per_tile, n_per_tile)],
                        out_hbm.at[c, pl.ds(s * n_per_tile, n_per_tile)])

    return pl.kernel(
        body,
        out_type=jax.ShapeDtypeStruct((NC, np_), jnp.float32),
        mesh=_mesh(),
        scratch_types=[
            pltpu.VMEM((rows_per_tile, CHUNK), jnp.int32),
            pltpu.VMEM((rows_per_tile, CHUNK), jnp.float32),
            pltpu.VMEM((n_per_tile,), jnp.float32),
            pltpu.VMEM_SHARED((np_,), jnp.float32),
            pltpu.SemaphoreType.DMA,
        ],
    )


PGC = 8
AGG_K0 = 80
AGG_K1 = 80


def _make_agg_kernel(np_, ep, d, k0=None, k1=None):
    total_chunks = ep // CHUNK
    if k0 is None:
        k0 = total_chunks // (2 * NS)
    if k1 is None:
        k1 = total_chunks // NS - k0
    assert NS * (k0 + k1) == total_chunks and k0 % (2 * PGC) == 0 and k1 % (2 * PGC) == 0, (k0, k1)
    n_per_tile = np_ // NS

    def body(y_hbm, src_hbm, dst_hbm, ew_hbm, out_hbm, *scratch):
        sp = scratch[0:2]
        dp = scratch[2:4]
        wp = scratch[4:6]
        rows = scratch[6:8]
        acc_sh = scratch[8]
        si = scratch[9:11]
        sg = scratch[11:13]

        c = lax.axis_index("c")
        s = lax.axis_index("s")
        trow = jnp.where(c == 0, s * k0, NS * k0 + s * k1)
        n_pairs = jnp.where(c == 0, k0 // (2 * PGC), k1 // (2 * PGC))

        def fire_page(prow, pb):
            pltpu.async_copy(src_hbm.at[pl.ds(prow, PGC)], sp[pb], si[pb])
            pltpu.async_copy(dst_hbm.at[pl.ds(prow, PGC)], dp[pb], si[pb])
            pltpu.async_copy(ew_hbm.at[pl.ds(prow, PGC)], wp[pb], si[pb])

        def wait_page(pb):
            pltpu.make_async_copy(src_hbm.at[pl.ds(0, PGC)], sp[pb], si[pb]).wait()
            pltpu.make_async_copy(dst_hbm.at[pl.ds(0, PGC)], dp[pb], si[pb]).wait()
            pltpu.make_async_copy(ew_hbm.at[pl.ds(0, PGC)], wp[pb], si[pb]).wait()

        def fire_gather(j, pb, rb):
            pltpu.async_copy(y_hbm.at[sp[pb].at[j]], rows[rb], sg[rb])

        def wait_gather(rb):
            pltpu.make_async_copy(y_hbm.at[sp[0].at[0]], rows[rb], sg[rb]).wait()

        def scale(j, pb, rb):
            def grp(g, _):
                wv = wp[pb][j, pl.ds(g * LANES, LANES)]
                for i in range(LANES):
                    w = wv[i]
                    row = g * LANES + i
                    for jj in range(d // LANES):
                        sl = rows[rb][row, pl.ds(jj * LANES, LANES)]
                        rows[rb][row, pl.ds(jj * LANES, LANES)] = sl * w
                return _
            lax.fori_loop(0, CHUNK // LANES, grp, None)

        def scatter(j, pb, rb):
            pltpu.sync_copy(rows[rb], acc_sh.at[dp[pb].at[j]], add=True)

        def zloop(i, _):
            for jj in range(d // LANES):
                rows[0][i, pl.ds(jj * LANES, LANES)] = jnp.zeros((LANES,), jnp.float32)
            return _
        lax.fori_loop(0, CHUNK, zloop, None)
        for b in range(n_per_tile // CHUNK):
            pltpu.sync_copy(rows[0], acc_sh.at[pl.ds(s * n_per_tile + b * CHUNK, CHUNK)])
        plsc.subcore_barrier()


        def do_page(prow, pb, pair_i, is_b):
            if not is_b:
                fire_page(prow + PGC, 1 - pb)
            else:
                @pl.when(pair_i <= n_pairs - 2)
                def _f():
                    fire_page(prow + PGC, 1 - pb)
            def cpair(jj, _):
                j0 = 2 * jj
                fire_gather(j0 + 1, pb, 1)
                wait_gather(0)
                scale(j0, pb, 0)
                scatter(j0, pb, 0)
                fire_gather(j0 + 2, pb, 0)
                wait_gather(1)
                scale(j0 + 1, pb, 1)
                scatter(j0 + 1, pb, 1)
                return _
            lax.fori_loop(0, PGC // 2 - 1, cpair, None)
            fire_gather(PGC - 1, pb, 1)
            wait_gather(0)
            scale(PGC - 2, pb, 0)
            scatter(PGC - 2, pb, 0)
            if not is_b:
                wait_page(1 - pb)
                fire_gather(0, 1 - pb, 0)
            else:
                @pl.when(pair_i <= n_pairs - 2)
                def _g():
                    wait_page(1 - pb)
                    fire_gather(0, 1 - pb, 0)
            wait_gather(1)
            scale(PGC - 1, pb, 1)
            scatter(PGC - 1, pb, 1)

        def pair_loop(pair_i, _):
            prow = trow + pair_i * 2 * PGC
            do_page(prow, 0, pair_i, False)
            do_page(prow + PGC, 1, pair_i, True)
            return _

        @pl.when(n_pairs > 0)
        def _edge_work():
            fire_page(trow, 0)
            wait_page(0)
            fire_gather(0, 0, 0)
            lax.fori_loop(0, n_pairs, pair_loop, None)
        plsc.subcore_barrier()

        pltpu.sync_copy(acc_sh.at[pl.ds(s * n_per_tile, n_per_tile)],
                        out_hbm.at[c, pl.ds(s * n_per_tile, n_per_tile)])

    return pl.kernel(
        body,
        out_type=jax.ShapeDtypeStruct((NC, np_, d), jnp.float32),
        mesh=_mesh(),
        scratch_types=(
            [pltpu.VMEM((PGC, CHUNK), jnp.int32) for _ in range(4)]
            + [pltpu.VMEM((PGC, CHUNK), jnp.float32) for _ in range(2)]
            + [pltpu.VMEM((CHUNK, d), jnp.float32) for _ in range(2)]
            + [pltpu.VMEM_SHARED((np_, d), jnp.float32)]
            + [pltpu.SemaphoreType.DMA for _ in range(4)]
        ),
    )


def _prep_body(degp_ref, x_ref, dinv_ref, y_ref):
    deg = degp_ref[0] + degp_ref[1] + 1.0
    dv = jnp.where(deg > 0, lax.rsqrt(deg), 0.0)
    dinv_ref[...] = dv
    y_ref[...] = dv * x_ref[...]


def _mid_body(aggp_ref, x_ref, dinv_ref, w1_ref, b1_ref, w2_ref, t_ref, y2_ref):
    dv = dinv_ref[...]
    ax = dv * (aggp_ref[0] + aggp_ref[1]) + (dv * dv) * x_ref[...]
    h = jnp.maximum(
        jnp.dot(ax, w1_ref[...], preferred_element_type=jnp.float32) + b1_ref[...],
        0.0)
    t = jnp.dot(h, w2_ref[...], preferred_element_type=jnp.float32)
    t_ref[...] = t
    y2_ref[...] = dv * t


def _final_body(aggp_ref, t_ref, dinv_ref, b2_ref, out_ref):
    dv = dinv_ref[...]
    out_ref[...] = (dv * (aggp_ref[0] + aggp_ref[1])
                    + (dv * dv) * t_ref[...] + b2_ref[...])


def kernel(x, edge_index, edge_weight, W1, b1, W2, b2):
    n, d_in = x.shape
    d_hid = W1.shape[1]
    d_out = W2.shape[1]
    e = edge_weight.shape[0]

    np_ = _round_up(n, NS * CHUNK)
    ep = _round_up(e, NC * NS * CHUNK * 2 * PGC)

    src = edge_index[0].astype(jnp.int32)
    dst = edge_index[1].astype(jnp.int32)
    ew = edge_weight.astype(jnp.float32)
    pad_e = ep - e
    pad_idx = jnp.arange(pad_e, dtype=jnp.int32)
    src = jnp.concatenate([src, pad_idx % n]).reshape(ep // CHUNK, CHUNK)
    dst = jnp.concatenate([dst, pad_idx % np_]).reshape(ep // CHUNK, CHUNK)
    ew = jnp.concatenate([ew, jnp.zeros((pad_e,), jnp.float32)]).reshape(ep // CHUNK, CHUNK)
    xp = jnp.concatenate([x, jnp.zeros((np_ - n, d_in), x.dtype)])

    degp = _make_deg_kernel(np_, ep)(dst, ew)

    r = 512
    grid = (np_ // r,)
    dinv, y = pl.pallas_call(
        _prep_body,
        grid=grid,
        in_specs=[
            pl.BlockSpec((NC, r, 1), lambda i: (0, i, 0)),
            pl.BlockSpec((r, d_in), lambda i: (i, 0)),
        ],
        out_specs=[
            pl.BlockSpec((r, 1), lambda i: (i, 0)),
            pl.BlockSpec((r, d_in), lambda i: (i, 0)),
        ],
        out_shape=[
            jax.ShapeDtypeStruct((np_, 1), jnp.float32),
            jax.ShapeDtypeStruct((np_, d_in), jnp.float32),
        ],
    )(degp.reshape(NC, np_, 1), xp)

    agg1 = _make_agg_kernel(np_, ep, d_in, AGG_K0, AGG_K1)(y, src, dst, ew)

    t, y2 = pl.pallas_call(
        _mid_body,
        grid=grid,
        in_specs=[
            pl.BlockSpec((NC, r, d_in), lambda i: (0, i, 0)),
            pl.BlockSpec((r, d_in), lambda i: (i, 0)),
            pl.BlockSpec((r, 1), lambda i: (i, 0)),
            pl.BlockSpec((d_in, d_hid), lambda i: (0, 0)),
            pl.BlockSpec((1, d_hid), lambda i: (0, 0)),
            pl.BlockSpec((d_hid, d_out), lambda i: (0, 0)),
        ],
        out_specs=[
            pl.BlockSpec((r, d_out), lambda i: (i, 0)),
            pl.BlockSpec((r, d_out), lambda i: (i, 0)),
        ],
        out_shape=[
            jax.ShapeDtypeStruct((np_, d_out), jnp.float32),
            jax.ShapeDtypeStruct((np_, d_out), jnp.float32),
        ],
    )(agg1, xp, dinv, W1, b1.reshape(1, d_hid), W2)

    agg2 = _make_agg_kernel(np_, ep, d_out, AGG_K0, AGG_K1)(y2, src, dst, ew)

    out = pl.pallas_call(
        _final_body,
        grid=grid,
        in_specs=[
            pl.BlockSpec((NC, r, d_out), lambda i: (0, i, 0)),
            pl.BlockSpec((r, d_out), lambda i: (i, 0)),
            pl.BlockSpec((r, 1), lambda i: (i, 0)),
            pl.BlockSpec((1, d_out), lambda i: (0, 0)),
        ],
        out_specs=pl.BlockSpec((r, d_out), lambda i: (i, 0)),
        out_shape=jax.ShapeDtypeStruct((np_, d_out), jnp.float32),
    )(agg2, t, dinv, b2.reshape(1, d_out))

    return out[:n]
